# Initial kernel scaffold; baseline (speedup 1.0000x reference)
import functools

import jax
import jax.numpy as jnp
from jax import lax
from jax.experimental import pallas as pl
from jax.experimental.pallas import tpu as pltpu
from jax.experimental.pallas import tpu_sc as plsc

N_NODES = 10000
N_EDGES = 320000
DIM = 128

NC = 2
NS = 16
NW = NC * NS

CK = 80
EDGES_PER_TILE = N_EDGES // NW
CHUNKS_PER_TILE = EDGES_PER_TILE // CK
ROWS_PER_TILE = N_NODES // NS
RB = 125
DEGW = 16


def _make_sc_aggregate(compute_deg: bool):
  mesh = plsc.VectorSubcoreMesh(
      core_axis_name="c", subcore_axis_name="s", num_cores=NC, num_subcores=NS)

  out_type = [jax.ShapeDtypeStruct((NC, N_NODES, DIM), jnp.float32)]
  if compute_deg:
    out_type.append(jax.ShapeDtypeStruct((NC, N_NODES, DEGW), jnp.float32))

  scratch = [
      pltpu.VMEM((CHUNKS_PER_TILE, CK), jnp.int32),
      pltpu.VMEM((CHUNKS_PER_TILE, CK), jnp.int32),
      pltpu.VMEM((CK, DIM), jnp.float32),
      pltpu.VMEM((RB, DIM), jnp.float32),
      pltpu.VMEM((RB, DEGW), jnp.float32),
      pltpu.VMEM((CK, DEGW), jnp.float32),
      pltpu.VMEM_SHARED((N_NODES, DIM), jnp.float32),
      pltpu.VMEM_SHARED((N_NODES, DEGW), jnp.float32),
      pltpu.SemaphoreType.DMA,
  ]

  def body(table_hbm, src_hbm, dst_hbm, *rest):
    if compute_deg:
      acc_out, deg_out = rest[0], rest[1]
      rest = rest[2:]
    else:
      acc_out = rest[0]
      deg_out = None
      rest = rest[1:]
    (src_v, dst_v, rows_v, zrow_v, zdeg_v, ones_v, acc_sh, deg_sh, gsem) = rest

    c = lax.axis_index("c")
    s = lax.axis_index("s")
    w = c * NS + s

    def fill_z(i, carry):
      for k in range(DIM // 16):
        zrow_v[i, pl.ds(k * 16, 16)] = jnp.zeros((16,), jnp.float32)
      zdeg_v[i, :] = jnp.zeros((16,), jnp.float32)
      return carry
    lax.fori_loop(0, RB, fill_z, 0)

    def fill_o(i, carry):
      ones_v[i, :] = jnp.ones((16,), jnp.float32)
      return carry
    lax.fori_loop(0, CK, fill_o, 0)

    row0 = s * ROWS_PER_TILE
    for k in range(ROWS_PER_TILE // RB):
      pltpu.sync_copy(zrow_v, acc_sh.at[pl.ds(row0 + k * RB, RB)])
      if compute_deg:
        pltpu.sync_copy(zdeg_v, deg_sh.at[pl.ds(row0 + k * RB, RB)])
    plsc.subcore_barrier()

    chunk0 = w * CHUNKS_PER_TILE
    pltpu.sync_copy(src_hbm.at[pl.ds(chunk0, CHUNKS_PER_TILE)], src_v)
    pltpu.sync_copy(dst_hbm.at[pl.ds(chunk0, CHUNKS_PER_TILE)], dst_v)

    def step(j, carry):
      pltpu.async_copy(table_hbm.at[src_v.at[j]], rows_v, gsem).wait()
      pltpu.sync_copy(rows_v, acc_sh.at[dst_v.at[j]], add=True)
      if compute_deg:
        pltpu.sync_copy(ones_v, deg_sh.at[dst_v.at[j]], add=True)
      return carry
    lax.fori_loop(0, CHUNKS_PER_TILE, step, 0)

    plsc.subcore_barrier()

    for k in range(ROWS_PER_TILE // RB):
      r = row0 + k * RB
      pltpu.sync_copy(acc_sh.at[pl.ds(r, RB)], zrow_v)
      pltpu.sync_copy(zrow_v, acc_out.at[c].at[pl.ds(r, RB)])
      if compute_deg:
        pltpu.sync_copy(deg_sh.at[pl.ds(r, RB)], zdeg_v)
        pltpu.sync_copy(zdeg_v, deg_out.at[c].at[pl.ds(r, RB)])

  return pl.kernel(body, out_type=out_type, mesh=mesh,
                   scratch_types=scratch, name="sc_sage_agg")


_sc_agg_l1 = _make_sc_aggregate(compute_deg=True)
_sc_agg_l2 = _make_sc_aggregate(compute_deg=False)


def _tc_layer_body(x_ref, p0_ref, p1_ref, d0_ref, d1_ref, ws_ref, wn_ref,
                   b_ref, g_ref, be_ref, o_ref):
  deg = d0_ref[...] + d1_ref[...]
  degc = jnp.maximum(deg[:, 0:1], 1.0)
  mean = (p0_ref[...] + p1_ref[...]) / degc
  h = (jnp.dot(x_ref[...], ws_ref[...], preferred_element_type=jnp.float32)
       + jnp.dot(mean, wn_ref[...], preferred_element_type=jnp.float32)
       + b_ref[...])
  m = jnp.mean(h, axis=0, keepdims=True)
  v = jnp.mean((h - m) * (h - m), axis=0, keepdims=True)
  hn = (h - m) * jax.lax.rsqrt(v + 1e-5) * g_ref[...] + be_ref[...]
  o_ref[...] = jnp.where(hn >= 0.0, hn, 0.01 * hn)


def _tc_layer(x, p0, p1, d0, d1, w_self, w_neigh, b, g, be):
  return pl.pallas_call(
      _tc_layer_body,
      out_shape=jax.ShapeDtypeStruct((N_NODES, DIM), jnp.float32),
  )(x, p0, p1, d0, d1, w_self, w_neigh,
    b.reshape(1, DIM), g.reshape(1, DIM), be.reshape(1, DIM))


def kernel(x, edge_index, W1_self, W1_neigh, b1, g1, be1,
           W2_self, W2_neigh, b2, g2, be2):
  src = edge_index[0].astype(jnp.int32).reshape(N_EDGES // CK, CK)
  dst = edge_index[1].astype(jnp.int32).reshape(N_EDGES // CK, CK)

  acc1, deg = _sc_agg_l1(x, src, dst)
  h1 = _tc_layer(x, acc1[0], acc1[1], deg[0], deg[1],
                 W1_self, W1_neigh, b1, g1, be1)
  acc2, = _sc_agg_l2(h1, src, dst)
  h2 = _tc_layer(h1, acc2[0], acc2[1], deg[0], deg[1],
                 W2_self, W2_neigh, b2, g2, be2)
  return h2


# speedup vs baseline: 5.0881x; 5.0881x over previous
import jax
import jax.numpy as jnp
from jax import lax
from jax.experimental import pallas as pl
from jax.experimental.pallas import tpu as pltpu
from jax.experimental.pallas import tpu_sc as plsc

N_NODES = 10000
N_PAD = 10240
N_EDGES = 320000
DIM = 128

NC = 2
NS = 16
NW = NC * NS

CK = 80
EDGES_PER_TILE = N_EDGES // NW
CHUNKS_PER_TILE = EDGES_PER_TILE // CK
ROWS_PER_TILE = N_PAD // NS
RB = 80

_MESH = plsc.VectorSubcoreMesh(
    core_axis_name="c", subcore_axis_name="s", num_cores=NC, num_subcores=NS)


def _fill(ref, n_rows, value):
  def body(i, carry):
    for k in range(DIM // 16):
      ref[i, pl.ds(k * 16, 16)] = jnp.full((16,), value, jnp.float32)
    return carry
  lax.fori_loop(0, n_rows, body, 0)


def _make_sc_feature():
  scratch = [
      pltpu.VMEM((2, CK), jnp.int32),
      pltpu.VMEM((CK, DIM), jnp.float32),
      pltpu.VMEM_SHARED((N_PAD, DIM), jnp.float32),
      pltpu.SemaphoreType.DMA,
  ]

  def body(table_hbm, ei_hbm, acc_out, idx_c, rows_v, acc_sh, gsem):
    c = lax.axis_index("c")
    s = lax.axis_index("s")
    w = c * NS + s
    row0 = s * ROWS_PER_TILE

    _fill(rows_v, CK, 0.0)
    for k in range(ROWS_PER_TILE // RB):
      pltpu.sync_copy(rows_v, acc_sh.at[pl.ds(row0 + k * RB, RB)])
    plsc.subcore_barrier()

    ei_w = ei_hbm.at[w]
    def step(j, carry):
      pltpu.sync_copy(ei_w.at[j], idx_c)
      pltpu.async_copy(table_hbm.at[idx_c.at[0]], rows_v, gsem).wait()
      pltpu.sync_copy(rows_v, acc_sh.at[idx_c.at[1]], add=True)
      return carry
    lax.fori_loop(0, CHUNKS_PER_TILE, step, 0)
    plsc.subcore_barrier()

    for k in range(ROWS_PER_TILE // RB):
      r = row0 + k * RB
      pltpu.sync_copy(acc_sh.at[pl.ds(r, RB)], rows_v)
      pltpu.sync_copy(rows_v, acc_out.at[c].at[pl.ds(r, RB)])

  return pl.kernel(
      body,
      out_type=jax.ShapeDtypeStruct((NC, N_PAD, DIM), jnp.float32),
      mesh=_MESH, scratch_types=scratch, name="sc_sage_feat")


def _make_sc_degree():
  scratch = [
      pltpu.VMEM((2, CK), jnp.int32),
      pltpu.VMEM((CK, DIM), jnp.float32),
      pltpu.VMEM_SHARED((N_PAD, DIM), jnp.float32),
  ]

  def body(ei_hbm, deg_out, idx_c, ones_v, deg_sh):
    c = lax.axis_index("c")
    s = lax.axis_index("s")
    w = c * NS + s
    row0 = s * ROWS_PER_TILE

    _fill(ones_v, CK, 0.0)
    for k in range(ROWS_PER_TILE // RB):
      pltpu.sync_copy(ones_v, deg_sh.at[pl.ds(row0 + k * RB, RB)])
    plsc.subcore_barrier()

    _fill(ones_v, CK, 1.0)
    ei_w = ei_hbm.at[w]
    def step(j, carry):
      pltpu.sync_copy(ei_w.at[j], idx_c)
      pltpu.sync_copy(ones_v, deg_sh.at[idx_c.at[1]], add=True)
      return carry
    lax.fori_loop(0, CHUNKS_PER_TILE, step, 0)
    plsc.subcore_barrier()

    for k in range(ROWS_PER_TILE // RB):
      r = row0 + k * RB
      pltpu.sync_copy(deg_sh.at[pl.ds(r, RB)], ones_v)
      pltpu.sync_copy(ones_v, deg_out.at[c].at[pl.ds(r, RB)])

  return pl.kernel(
      body,
      out_type=jax.ShapeDtypeStruct((NC, N_PAD, DIM), jnp.float32),
      mesh=_MESH, scratch_types=scratch, name="sc_sage_deg")


_sc_feat = _make_sc_feature()
_sc_deg = _make_sc_degree()


def _tc_body(x_ref, p0_ref, p1_ref, d0_ref, d1_ref, ws_ref, wn_ref,
             b_ref, g_ref, be_ref, o_ref):
  deg = d0_ref[:, 0:1] + d1_ref[:, 0:1]
  degc = jnp.maximum(deg, 1.0)
  mean = (p0_ref[...] + p1_ref[...]) / degc
  h = (jnp.dot(x_ref[...], ws_ref[...], preferred_element_type=jnp.float32)
       + jnp.dot(mean, wn_ref[...], preferred_element_type=jnp.float32)
       + b_ref[...])
  m = jnp.mean(h, axis=0, keepdims=True)
  v = jnp.mean((h - m) * (h - m), axis=0, keepdims=True)
  hn = (h - m) * jax.lax.rsqrt(v + 1e-5) * g_ref[...] + be_ref[...]
  o_ref[...] = jnp.where(hn >= 0.0, hn, 0.01 * hn)


def _tc_layer(x, p0, p1, d0, d1, w_self, w_neigh, b, g, be):
  return pl.pallas_call(
      _tc_body,
      out_shape=jax.ShapeDtypeStruct((N_NODES, DIM), jnp.float32),
  )(x, p0, p1, d0, d1, w_self, w_neigh,
    b.reshape(1, DIM), g.reshape(1, DIM), be.reshape(1, DIM))


def kernel(x, edge_index, W1_self, W1_neigh, b1, g1, be1,
           W2_self, W2_neigh, b2, g2, be2):
  ei = jnp.stack([
      edge_index[0].astype(jnp.int32).reshape(NW, CHUNKS_PER_TILE, CK),
      edge_index[1].astype(jnp.int32).reshape(NW, CHUNKS_PER_TILE, CK),
  ], axis=2)

  degp = _sc_deg(ei)
  d0, d1 = degp[0, :N_NODES], degp[1, :N_NODES]
  acc1 = _sc_feat(x, ei)
  h1 = _tc_layer(x, acc1[0, :N_NODES], acc1[1, :N_NODES], d0, d1,
                 W1_self, W1_neigh, b1, g1, be1)
  acc2 = _sc_feat(h1, ei)
  h2 = _tc_layer(h1, acc2[0, :N_NODES], acc2[1, :N_NODES], d0, d1,
                 W2_self, W2_neigh, b2, g2, be2)
  return h2


# speedup vs baseline: 7.7475x; 1.5227x over previous
import jax
import jax.numpy as jnp
from jax import lax
from jax.experimental import pallas as pl
from jax.experimental.pallas import tpu as pltpu
from jax.experimental.pallas import tpu_sc as plsc

N_NODES = 10000
N_PAD = 10240
N_EDGES = 320000
DIM = 128

NC = 2
NS = 16
NW = NC * NS

CK = 80
EDGES_PER_TILE = N_EDGES // NW
CHUNKS_PER_TILE = EDGES_PER_TILE // CK
U = 4
NGROUPS = CHUNKS_PER_TILE // U
ROWS_PER_TILE = N_PAD // NS
RB = 80

_MESH = plsc.VectorSubcoreMesh(
    core_axis_name="c", subcore_axis_name="s", num_cores=NC, num_subcores=NS)


def _fill(ref, n_rows, value):
  def body(i, carry):
    for k in range(DIM // 16):
      ref[i, pl.ds(k * 16, 16)] = jnp.full((16,), value, jnp.float32)
    return carry
  lax.fori_loop(0, n_rows, body, 0)


def _make_sc_feature():
  scratch = [
      pltpu.VMEM((U, 2, CK), jnp.int32),
      pltpu.VMEM((U, CK, DIM), jnp.float32),
      pltpu.VMEM_SHARED((N_PAD, DIM), jnp.float32),
  ] + [pltpu.SemaphoreType.DMA] * (2 * U)

  def body(table_hbm, ei_hbm, acc_out, idxg, rowsg, acc_sh, *sems):
    gsem, ssem = sems[:U], sems[U:]
    c = lax.axis_index("c")
    s = lax.axis_index("s")
    w = c * NS + s
    row0 = s * ROWS_PER_TILE

    _fill(rowsg.at[0], CK, 0.0)
    for k in range(ROWS_PER_TILE // RB):
      pltpu.sync_copy(rowsg.at[0], acc_sh.at[pl.ds(row0 + k * RB, RB)])
    plsc.subcore_barrier()

    ei_w = ei_hbm.at[w]

    def run_group(start, n):
      pltpu.sync_copy(ei_w.at[pl.ds(start, n)], idxg.at[pl.ds(0, n)])
      gds = [pltpu.async_copy(table_hbm.at[idxg.at[b].at[0]], rowsg.at[b],
                              gsem[b]) for b in range(n)]
      sds = []
      for b in range(n):
        gds[b].wait()
        sds.append(pltpu.async_copy(rowsg.at[b], acc_sh.at[idxg.at[b].at[1]],
                                    ssem[b], add=True))
      for d in sds:
        d.wait()

    def step(g, carry):
      run_group(g * U, U)
      return carry
    lax.fori_loop(0, NGROUPS, step, 0)
    for j in range(NGROUPS * U, CHUNKS_PER_TILE):
      run_group(j, 1)
    plsc.subcore_barrier()

    for k in range(ROWS_PER_TILE // RB):
      r = row0 + k * RB
      pltpu.sync_copy(acc_sh.at[pl.ds(r, RB)], rowsg.at[0])
      pltpu.sync_copy(rowsg.at[0], acc_out.at[c].at[pl.ds(r, RB)])

  return pl.kernel(
      body,
      out_type=jax.ShapeDtypeStruct((NC, N_PAD, DIM), jnp.float32),
      mesh=_MESH, scratch_types=scratch, name="sc_sage_feat")


def _make_sc_degree():
  scratch = [
      pltpu.VMEM((U, 2, CK), jnp.int32),
      pltpu.VMEM((CK, DIM), jnp.float32),
      pltpu.VMEM_SHARED((N_PAD, DIM), jnp.float32),
  ] + [pltpu.SemaphoreType.DMA] * U

  def body(ei_hbm, deg_out, idxg, ones_v, deg_sh, *ssem):
    c = lax.axis_index("c")
    s = lax.axis_index("s")
    w = c * NS + s
    row0 = s * ROWS_PER_TILE

    _fill(ones_v, CK, 0.0)
    for k in range(ROWS_PER_TILE // RB):
      pltpu.sync_copy(ones_v, deg_sh.at[pl.ds(row0 + k * RB, RB)])
    plsc.subcore_barrier()

    _fill(ones_v, CK, 1.0)
    ei_w = ei_hbm.at[w]

    def run_group(start, n):
      pltpu.sync_copy(ei_w.at[pl.ds(start, n)], idxg.at[pl.ds(0, n)])
      sds = [pltpu.async_copy(ones_v, deg_sh.at[idxg.at[b].at[1]],
                              ssem[b], add=True) for b in range(n)]
      for d in sds:
        d.wait()

    def step(g, carry):
      run_group(g * U, U)
      return carry
    lax.fori_loop(0, NGROUPS, step, 0)
    for j in range(NGROUPS * U, CHUNKS_PER_TILE):
      run_group(j, 1)
    plsc.subcore_barrier()

    for k in range(ROWS_PER_TILE // RB):
      r = row0 + k * RB
      pltpu.sync_copy(deg_sh.at[pl.ds(r, RB)], ones_v)
      pltpu.sync_copy(ones_v, deg_out.at[c].at[pl.ds(r, RB)])

  return pl.kernel(
      body,
      out_type=jax.ShapeDtypeStruct((NC, N_PAD, DIM), jnp.float32),
      mesh=_MESH, scratch_types=scratch, name="sc_sage_deg")


_sc_feat = _make_sc_feature()
_sc_deg = _make_sc_degree()


def _tc_body(x_ref, p0_ref, p1_ref, d0_ref, d1_ref, ws_ref, wn_ref,
             b_ref, g_ref, be_ref, o_ref):
  deg = d0_ref[:, 0:1] + d1_ref[:, 0:1]
  degc = jnp.maximum(deg, 1.0)
  mean = (p0_ref[...] + p1_ref[...]) / degc
  h = (jnp.dot(x_ref[...], ws_ref[...], preferred_element_type=jnp.float32)
       + jnp.dot(mean, wn_ref[...], preferred_element_type=jnp.float32)
       + b_ref[...])
  m = jnp.mean(h, axis=0, keepdims=True)
  v = jnp.mean((h - m) * (h - m), axis=0, keepdims=True)
  hn = (h - m) * jax.lax.rsqrt(v + 1e-5) * g_ref[...] + be_ref[...]
  o_ref[...] = jnp.where(hn >= 0.0, hn, 0.01 * hn)


def _tc_layer(x, p0, p1, d0, d1, w_self, w_neigh, b, g, be):
  return pl.pallas_call(
      _tc_body,
      out_shape=jax.ShapeDtypeStruct((N_NODES, DIM), jnp.float32),
  )(x, p0, p1, d0, d1, w_self, w_neigh,
    b.reshape(1, DIM), g.reshape(1, DIM), be.reshape(1, DIM))


def kernel(x, edge_index, W1_self, W1_neigh, b1, g1, be1,
           W2_self, W2_neigh, b2, g2, be2):
  ei = jnp.stack([
      edge_index[0].astype(jnp.int32).reshape(NW, CHUNKS_PER_TILE, CK),
      edge_index[1].astype(jnp.int32).reshape(NW, CHUNKS_PER_TILE, CK),
  ], axis=2)

  degp = _sc_deg(ei)
  d0, d1 = degp[0, :N_NODES], degp[1, :N_NODES]
  acc1 = _sc_feat(x, ei)
  h1 = _tc_layer(x, acc1[0, :N_NODES], acc1[1, :N_NODES], d0, d1,
                 W1_self, W1_neigh, b1, g1, be1)
  acc2 = _sc_feat(h1, ei)
  h2 = _tc_layer(h1, acc2[0, :N_NODES], acc2[1, :N_NODES], d0, d1,
                 W2_self, W2_neigh, b2, g2, be2)
  return h2


# speedup vs baseline: 8.9373x; 1.1536x over previous
import jax
import jax.numpy as jnp
from jax import lax
from jax.experimental import pallas as pl
from jax.experimental.pallas import tpu as pltpu
from jax.experimental.pallas import tpu_sc as plsc

N_NODES = 10000
N_PAD = 10240
N_EDGES = 320000
DIM = 128

NC = 2
NS = 16
NW = NC * NS

CK = 80
EDGES_PER_TILE = N_EDGES // NW
CHUNKS_PER_TILE = EDGES_PER_TILE // CK
U = 4
NGROUPS = CHUNKS_PER_TILE // U
ROWS_PER_TILE = N_PAD // NS
RB = 80

_MESH = plsc.VectorSubcoreMesh(
    core_axis_name="c", subcore_axis_name="s", num_cores=NC, num_subcores=NS)


def _fill(ref, n_rows, value):
  def body(i, carry):
    for k in range(DIM // 16):
      ref[i, pl.ds(k * 16, 16)] = jnp.full((16,), value, jnp.float32)
    return carry
  lax.fori_loop(0, n_rows, body, 0)


def _make_sc_feature():
  scratch = [
      pltpu.VMEM((2, U, 2, CK), jnp.int32),
      pltpu.VMEM((U, CK, DIM), jnp.float32),
      pltpu.VMEM_SHARED((N_PAD, DIM), jnp.float32),
  ] + [pltpu.SemaphoreType.DMA] * (2 * U + 1)

  def body(table_hbm, ei_hbm, acc_out, idxg2, rowsg, acc_sh, *sems):
    gsem, ssem, isem = sems[:U], sems[U:2 * U], sems[2 * U]
    c = lax.axis_index("c")
    s = lax.axis_index("s")
    w = c * NS + s
    row0 = s * ROWS_PER_TILE

    _fill(rowsg.at[0], CK, 0.0)
    for k in range(ROWS_PER_TILE // RB):
      pltpu.sync_copy(rowsg.at[0], acc_sh.at[pl.ds(row0 + k * RB, RB)])
    plsc.subcore_barrier()

    ei_w = ei_hbm.at[w]

    def do_chunks(idxg, n):
      gds = [pltpu.async_copy(table_hbm.at[idxg.at[b].at[0]], rowsg.at[b],
                              gsem[b]) for b in range(n)]
      sds = []
      for b in range(n):
        gds[b].wait()
        sds.append(pltpu.async_copy(rowsg.at[b], acc_sh.at[idxg.at[b].at[1]],
                                    ssem[b], add=True))
      for d in sds:
        d.wait()

    pltpu.async_copy(ei_w.at[pl.ds(0, U)], idxg2.at[0], isem)
    def step(g, carry):
      slot = lax.rem(g, 2)
      pltpu.make_async_copy(ei_w.at[pl.ds(g * U, U)], idxg2.at[slot],
                            isem).wait()
      @pl.when(g + 1 < NGROUPS)
      def _():
        pltpu.async_copy(ei_w.at[pl.ds((g + 1) * U, U)],
                         idxg2.at[lax.rem(g + 1, 2)], isem)
      do_chunks(idxg2.at[slot], U)
      return carry
    lax.fori_loop(0, NGROUPS, step, 0)
    for j in range(NGROUPS * U, CHUNKS_PER_TILE):
      pltpu.sync_copy(ei_w.at[pl.ds(j, 1)], idxg2.at[0].at[pl.ds(0, 1)])
      do_chunks(idxg2.at[0], 1)
    plsc.subcore_barrier()

    for k in range(ROWS_PER_TILE // RB):
      r = row0 + k * RB
      pltpu.sync_copy(acc_sh.at[pl.ds(r, RB)], rowsg.at[0])
      pltpu.sync_copy(rowsg.at[0], acc_out.at[c].at[pl.ds(r, RB)])

  return pl.kernel(
      body,
      out_type=jax.ShapeDtypeStruct((NC, N_PAD, DIM), jnp.float32),
      mesh=_MESH, scratch_types=scratch, name="sc_sage_feat")


def _make_sc_degree():
  scratch = [
      pltpu.VMEM((2, U, 2, CK), jnp.int32),
      pltpu.VMEM((CK, DIM), jnp.float32),
      pltpu.VMEM_SHARED((N_PAD, DIM), jnp.float32),
  ] + [pltpu.SemaphoreType.DMA] * (U + 1)

  def body(ei_hbm, deg_out, idxg2, ones_v, deg_sh, *sems):
    ssem, isem = sems[:U], sems[U]
    c = lax.axis_index("c")
    s = lax.axis_index("s")
    w = c * NS + s
    row0 = s * ROWS_PER_TILE

    _fill(ones_v, CK, 0.0)
    for k in range(ROWS_PER_TILE // RB):
      pltpu.sync_copy(ones_v, deg_sh.at[pl.ds(row0 + k * RB, RB)])
    plsc.subcore_barrier()

    _fill(ones_v, CK, 1.0)
    ei_w = ei_hbm.at[w]

    def do_chunks(idxg, n):
      sds = [pltpu.async_copy(ones_v, deg_sh.at[idxg.at[b].at[1]],
                              ssem[b], add=True) for b in range(n)]
      for d in sds:
        d.wait()

    pltpu.async_copy(ei_w.at[pl.ds(0, U)], idxg2.at[0], isem)
    def step(g, carry):
      slot = lax.rem(g, 2)
      pltpu.make_async_copy(ei_w.at[pl.ds(g * U, U)], idxg2.at[slot],
                            isem).wait()
      @pl.when(g + 1 < NGROUPS)
      def _():
        pltpu.async_copy(ei_w.at[pl.ds((g + 1) * U, U)],
                         idxg2.at[lax.rem(g + 1, 2)], isem)
      do_chunks(idxg2.at[slot], U)
      return carry
    lax.fori_loop(0, NGROUPS, step, 0)
    for j in range(NGROUPS * U, CHUNKS_PER_TILE):
      pltpu.sync_copy(ei_w.at[pl.ds(j, 1)], idxg2.at[0].at[pl.ds(0, 1)])
      do_chunks(idxg2.at[0], 1)
    plsc.subcore_barrier()

    for k in range(ROWS_PER_TILE // RB):
      r = row0 + k * RB
      pltpu.sync_copy(deg_sh.at[pl.ds(r, RB)], ones_v)
      pltpu.sync_copy(ones_v, deg_out.at[c].at[pl.ds(r, RB)])

  return pl.kernel(
      body,
      out_type=jax.ShapeDtypeStruct((NC, N_PAD, DIM), jnp.float32),
      mesh=_MESH, scratch_types=scratch, name="sc_sage_deg")


_sc_feat = _make_sc_feature()
_sc_deg = _make_sc_degree()


def _tc_body(x_ref, p_ref, d0_ref, d1_ref, ws_ref, wn_ref,
             b_ref, g_ref, be_ref, o_ref):
  deg = d0_ref[...] + d1_ref[...]
  degc = jnp.maximum(deg, 1.0)
  mean = (p_ref[0, :N_NODES, :] + p_ref[1, :N_NODES, :]) / degc
  h = (jnp.dot(x_ref[...], ws_ref[...], preferred_element_type=jnp.float32)
       + jnp.dot(mean, wn_ref[...], preferred_element_type=jnp.float32)
       + b_ref[...])
  m = jnp.mean(h, axis=0, keepdims=True)
  v = jnp.mean((h - m) * (h - m), axis=0, keepdims=True)
  hn = (h - m) * jax.lax.rsqrt(v + 1e-5) * g_ref[...] + be_ref[...]
  o_ref[...] = jnp.where(hn >= 0.0, hn, 0.01 * hn)


def _tc_layer(x, p, d0, d1, w_self, w_neigh, b, g, be):
  return pl.pallas_call(
      _tc_body,
      out_shape=jax.ShapeDtypeStruct((N_NODES, DIM), jnp.float32),
  )(x, p, d0, d1, w_self, w_neigh,
    b.reshape(1, DIM), g.reshape(1, DIM), be.reshape(1, DIM))


def kernel(x, edge_index, W1_self, W1_neigh, b1, g1, be1,
           W2_self, W2_neigh, b2, g2, be2):
  ei = jnp.stack([
      edge_index[0].astype(jnp.int32).reshape(NW, CHUNKS_PER_TILE, CK),
      edge_index[1].astype(jnp.int32).reshape(NW, CHUNKS_PER_TILE, CK),
  ], axis=2)

  degp = _sc_deg(ei)
  d0 = degp[0, :N_NODES, 0:1]
  d1 = degp[1, :N_NODES, 0:1]
  acc1 = _sc_feat(x, ei)
  h1 = _tc_layer(x, acc1, d0, d1, W1_self, W1_neigh, b1, g1, be1)
  acc2 = _sc_feat(h1, ei)
  h2 = _tc_layer(h1, acc2, d0, d1, W2_self, W2_neigh, b2, g2, be2)
  return h2


# speedup vs baseline: 10.3656x; 1.1598x over previous
import jax
import jax.numpy as jnp
from jax import lax
from jax.experimental import pallas as pl
from jax.experimental.pallas import tpu as pltpu
from jax.experimental.pallas import tpu_sc as plsc

N_NODES = 10000
N_PAD = 10240
N_EDGES = 320000
DIM = 128

NC = 2
NS = 16
NW = NC * NS

CK = 80
EDGES_PER_TILE = N_EDGES // NW
CHUNKS_PER_TILE = EDGES_PER_TILE // CK
U = 4
NGROUPS = CHUNKS_PER_TILE // U
ROWS_PER_TILE = N_PAD // NS
RB = 80

_MESH = plsc.VectorSubcoreMesh(
    core_axis_name="c", subcore_axis_name="s", num_cores=NC, num_subcores=NS)


def _fill(ref, n_rows, value):
  def body(i, carry):
    for k in range(DIM // 16):
      ref[i, pl.ds(k * 16, 16)] = jnp.full((16,), value, jnp.float32)
    return carry
  lax.fori_loop(0, n_rows, body, 0)


def _make_sc_feature():
  scratch = [
      pltpu.VMEM((2, U, 2, CK), jnp.int32),
      pltpu.VMEM((U, CK, DIM), jnp.float32),
      pltpu.VMEM_SHARED((N_PAD, DIM), jnp.float32),
  ] + [pltpu.SemaphoreType.DMA] * (2 * U + 1)

  def body(table_hbm, ei_hbm, acc_out, idxg2, rowsg, acc_sh, *sems):
    gsem, ssem, isem = sems[:U], sems[U:2 * U], sems[2 * U]
    c = lax.axis_index("c")
    s = lax.axis_index("s")
    w = c * NS + s
    row0 = s * ROWS_PER_TILE

    _fill(rowsg.at[0], CK, 0.0)
    for k in range(ROWS_PER_TILE // RB):
      pltpu.sync_copy(rowsg.at[0], acc_sh.at[pl.ds(row0 + k * RB, RB)])
    plsc.subcore_barrier()

    ei_w = ei_hbm.at[w]

    pltpu.async_copy(ei_w.at[pl.ds(0, U)], idxg2.at[0], isem)
    def step(g, carry):
      slot = lax.rem(g, 2)
      pltpu.make_async_copy(ei_w.at[pl.ds(g * U, U)], idxg2.at[slot],
                            isem).wait()
      idxg = idxg2.at[slot]
      gds = []
      for b in range(U):
        @pl.when(g > 0)
        def _(b=b):
          pltpu.make_async_copy(rowsg.at[b], acc_sh.at[idxg.at[b].at[1]],
                                ssem[b]).wait()
        gds.append(pltpu.async_copy(table_hbm.at[idxg.at[b].at[0]],
                                    rowsg.at[b], gsem[b]))
      @pl.when(g + 1 < NGROUPS)
      def _():
        pltpu.async_copy(ei_w.at[pl.ds((g + 1) * U, U)],
                         idxg2.at[lax.rem(g + 1, 2)], isem)
      for b in range(U):
        gds[b].wait()
        pltpu.async_copy(rowsg.at[b], acc_sh.at[idxg.at[b].at[1]],
                         ssem[b], add=True)
      return carry
    lax.fori_loop(0, NGROUPS, step, 0)
    for b in range(U):
      pltpu.make_async_copy(rowsg.at[b], acc_sh.at[idxg2.at[0].at[b].at[1]],
                            ssem[b]).wait()
    for j in range(NGROUPS * U, CHUNKS_PER_TILE):
      pltpu.sync_copy(ei_w.at[pl.ds(j, 1)], idxg2.at[0].at[pl.ds(0, 1)])
      lidx = idxg2.at[0].at[0]
      pltpu.async_copy(table_hbm.at[lidx.at[0]], rowsg.at[0], gsem[0]).wait()
      pltpu.sync_copy(rowsg.at[0], acc_sh.at[lidx.at[1]], add=True)
    plsc.subcore_barrier()

    for k in range(ROWS_PER_TILE // RB):
      r = row0 + k * RB
      pltpu.sync_copy(acc_sh.at[pl.ds(r, RB)], rowsg.at[0])
      pltpu.sync_copy(rowsg.at[0], acc_out.at[c].at[pl.ds(r, RB)])

  return pl.kernel(
      body,
      out_type=jax.ShapeDtypeStruct((NC, N_PAD, DIM), jnp.float32),
      mesh=_MESH, scratch_types=scratch, name="sc_sage_feat")


def _make_sc_degree():
  scratch = [
      pltpu.VMEM((2, U, 2, CK), jnp.int32),
      pltpu.VMEM((CK, DIM), jnp.float32),
      pltpu.VMEM_SHARED((N_PAD, DIM), jnp.float32),
  ] + [pltpu.SemaphoreType.DMA] * (U + 1)

  def body(ei_hbm, deg_out, idxg2, ones_v, deg_sh, *sems):
    ssem, isem = sems[:U], sems[U]
    c = lax.axis_index("c")
    s = lax.axis_index("s")
    w = c * NS + s
    row0 = s * ROWS_PER_TILE

    _fill(ones_v, CK, 0.0)
    for k in range(ROWS_PER_TILE // RB):
      pltpu.sync_copy(ones_v, deg_sh.at[pl.ds(row0 + k * RB, RB)])
    plsc.subcore_barrier()

    _fill(ones_v, CK, 1.0)
    ei_w = ei_hbm.at[w]

    pltpu.async_copy(ei_w.at[pl.ds(0, U)], idxg2.at[0], isem)
    def step(g, carry):
      slot = lax.rem(g, 2)
      pltpu.make_async_copy(ei_w.at[pl.ds(g * U, U)], idxg2.at[slot],
                            isem).wait()
      idxg = idxg2.at[slot]
      for b in range(U):
        @pl.when(g > 0)
        def _(b=b):
          pltpu.make_async_copy(ones_v, deg_sh.at[idxg.at[b].at[1]],
                                ssem[b]).wait()
        pltpu.async_copy(ones_v, deg_sh.at[idxg.at[b].at[1]],
                         ssem[b], add=True)
      @pl.when(g + 1 < NGROUPS)
      def _():
        pltpu.async_copy(ei_w.at[pl.ds((g + 1) * U, U)],
                         idxg2.at[lax.rem(g + 1, 2)], isem)
      return carry
    lax.fori_loop(0, NGROUPS, step, 0)
    for b in range(U):
      pltpu.make_async_copy(ones_v, deg_sh.at[idxg2.at[0].at[b].at[1]],
                            ssem[b]).wait()
    for j in range(NGROUPS * U, CHUNKS_PER_TILE):
      pltpu.sync_copy(ei_w.at[pl.ds(j, 1)], idxg2.at[0].at[pl.ds(0, 1)])
      pltpu.sync_copy(ones_v, deg_sh.at[idxg2.at[0].at[0].at[1]], add=True)
    plsc.subcore_barrier()

    for k in range(ROWS_PER_TILE // RB):
      r = row0 + k * RB
      pltpu.sync_copy(deg_sh.at[pl.ds(r, RB)], ones_v)
      pltpu.sync_copy(ones_v, deg_out.at[c].at[pl.ds(r, RB)])

  return pl.kernel(
      body,
      out_type=jax.ShapeDtypeStruct((NC, N_PAD, DIM), jnp.float32),
      mesh=_MESH, scratch_types=scratch, name="sc_sage_deg")


_sc_feat = _make_sc_feature()
_sc_deg = _make_sc_degree()


def _tc_body(x_ref, p_ref, d0_ref, d1_ref, ws_ref, wn_ref,
             b_ref, g_ref, be_ref, o_ref):
  deg = d0_ref[...] + d1_ref[...]
  degc = jnp.maximum(deg, 1.0)
  mean = (p_ref[0, :N_NODES, :] + p_ref[1, :N_NODES, :]) / degc
  h = (jnp.dot(x_ref[...], ws_ref[...], preferred_element_type=jnp.float32)
       + jnp.dot(mean, wn_ref[...], preferred_element_type=jnp.float32)
       + b_ref[...])
  m = jnp.mean(h, axis=0, keepdims=True)
  v = jnp.mean((h - m) * (h - m), axis=0, keepdims=True)
  hn = (h - m) * jax.lax.rsqrt(v + 1e-5) * g_ref[...] + be_ref[...]
  o_ref[...] = jnp.where(hn >= 0.0, hn, 0.01 * hn)


def _tc_layer(x, p, d0, d1, w_self, w_neigh, b, g, be):
  return pl.pallas_call(
      _tc_body,
      out_shape=jax.ShapeDtypeStruct((N_NODES, DIM), jnp.float32),
  )(x, p, d0, d1, w_self, w_neigh,
    b.reshape(1, DIM), g.reshape(1, DIM), be.reshape(1, DIM))


def kernel(x, edge_index, W1_self, W1_neigh, b1, g1, be1,
           W2_self, W2_neigh, b2, g2, be2):
  ei = jnp.stack([
      edge_index[0].astype(jnp.int32).reshape(NW, CHUNKS_PER_TILE, CK),
      edge_index[1].astype(jnp.int32).reshape(NW, CHUNKS_PER_TILE, CK),
  ], axis=2)

  degp = _sc_deg(ei)
  d0 = degp[0, :N_NODES, 0:1]
  d1 = degp[1, :N_NODES, 0:1]
  acc1 = _sc_feat(x, ei)
  h1 = _tc_layer(x, acc1, d0, d1, W1_self, W1_neigh, b1, g1, be1)
  acc2 = _sc_feat(h1, ei)
  h2 = _tc_layer(h1, acc2, d0, d1, W2_self, W2_neigh, b2, g2, be2)
  return h2


# speedup vs baseline: 10.5350x; 1.0163x over previous
import jax
import jax.numpy as jnp
from jax import lax
from jax.experimental import pallas as pl
from jax.experimental.pallas import tpu as pltpu
from jax.experimental.pallas import tpu_sc as plsc

N_NODES = 10000
N_PAD = 10240
N_EDGES = 320000
DIM = 128

NC = 2
NS = 16
NW = NC * NS

CK = 80
EDGES_PER_TILE = N_EDGES // NW
CHUNKS_PER_TILE = EDGES_PER_TILE // CK
U = 4
NGROUPS = CHUNKS_PER_TILE // U
ROWS_PER_TILE = N_PAD // NS
RB = 80

_MESH = plsc.VectorSubcoreMesh(
    core_axis_name="c", subcore_axis_name="s", num_cores=NC, num_subcores=NS)


def _fill(ref, n_rows, value):
  def body(i, carry):
    for k in range(DIM // 16):
      ref[i, pl.ds(k * 16, 16)] = jnp.full((16,), value, jnp.float32)
    return carry
  lax.fori_loop(0, n_rows, body, 0)


def _make_sc_pass(compute_deg: bool):
  out_type = [jax.ShapeDtypeStruct((NC, N_PAD, DIM), jnp.float32)]
  if compute_deg:
    out_type.append(jax.ShapeDtypeStruct((NC, N_PAD, DIM), jnp.float32))

  scratch = [
      pltpu.VMEM((2, U, 2, CK), jnp.int32),
      pltpu.VMEM((U, CK, DIM), jnp.float32),
      pltpu.VMEM_SHARED((N_PAD, DIM), jnp.float32),
  ] + [pltpu.SemaphoreType.DMA] * (2 * U + 3)

  def body(table_hbm, ei_hbm, *rest):
    if compute_deg:
      acc_out, deg_out = rest[0], rest[1]
      rest = rest[2:]
    else:
      acc_out, deg_out = rest[0], None
      rest = rest[1:]
    idxg2, rowsg, acc_sh = rest[:3]
    sems = rest[3:]
    gsem, ssem = sems[:U], sems[U:2 * U]
    isem = sems[2 * U]
    wsem = sems[2 * U + 1:2 * U + 3]

    c = lax.axis_index("c")
    s = lax.axis_index("s")
    w = c * NS + s
    row0 = s * ROWS_PER_TILE
    ei_w = ei_hbm.at[w]

    def zero_stripe():
      _fill(rowsg.at[0], CK, 0.0)
      for k in range(ROWS_PER_TILE // RB):
        pltpu.sync_copy(rowsg.at[0], acc_sh.at[pl.ds(row0 + k * RB, RB)])

    def epilogue(out_hbm):
      wds = [None, None]
      for k in range(ROWS_PER_TILE // RB):
        b = k % 2
        if wds[b] is not None:
          wds[b].wait()
        r = row0 + k * RB
        pltpu.sync_copy(acc_sh.at[pl.ds(r, RB)], rowsg.at[1 + b])
        wds[b] = pltpu.async_copy(rowsg.at[1 + b],
                                  out_hbm.at[c].at[pl.ds(r, RB)], wsem[b])
      for d in wds:
        d.wait()

    def scatter_loop(do_gather):
      src_buf = lambda b: rowsg.at[b] if do_gather else rowsg.at[0]
      pltpu.async_copy(ei_w.at[pl.ds(0, U)], idxg2.at[0], isem)

      def step(g, carry):
        slot = lax.rem(g, 2)
        pltpu.make_async_copy(ei_w.at[pl.ds(g * U, U)], idxg2.at[slot],
                              isem).wait()
        idxg = idxg2.at[slot]
        gds = []
        for b in range(U):
          @pl.when(g > 0)
          def _(b=b):
            pltpu.make_async_copy(src_buf(b), acc_sh.at[idxg.at[b].at[1]],
                                  ssem[b]).wait()
          if do_gather:
            gds.append(pltpu.async_copy(table_hbm.at[idxg.at[b].at[0]],
                                        rowsg.at[b], gsem[b]))
        @pl.when(g + 1 < NGROUPS)
        def _():
          pltpu.async_copy(ei_w.at[pl.ds((g + 1) * U, U)],
                           idxg2.at[lax.rem(g + 1, 2)], isem)
        for b in range(U):
          if do_gather:
            gds[b].wait()
          pltpu.async_copy(src_buf(b), acc_sh.at[idxg.at[b].at[1]],
                           ssem[b], add=True)
        return carry
      lax.fori_loop(0, NGROUPS, step, 0)

      for b in range(U):
        pltpu.make_async_copy(src_buf(b), acc_sh.at[idxg2.at[0].at[b].at[1]],
                              ssem[b]).wait()
      for j in range(NGROUPS * U, CHUNKS_PER_TILE):
        pltpu.sync_copy(ei_w.at[pl.ds(j, 1)], idxg2.at[0].at[pl.ds(0, 1)])
        lidx = idxg2.at[0].at[0]
        if do_gather:
          pltpu.async_copy(table_hbm.at[lidx.at[0]], rowsg.at[0],
                           gsem[0]).wait()
        pltpu.sync_copy(src_buf(0), acc_sh.at[lidx.at[1]], add=True)

    if compute_deg:
      zero_stripe()
      plsc.subcore_barrier()
      _fill(rowsg.at[0], CK, 1.0)
      scatter_loop(do_gather=False)
      plsc.subcore_barrier()
      epilogue(deg_out)

    zero_stripe()
    plsc.subcore_barrier()
    scatter_loop(do_gather=True)
    plsc.subcore_barrier()
    epilogue(acc_out)

  return pl.kernel(body, out_type=out_type, mesh=_MESH,
                   scratch_types=scratch, name="sc_sage_agg")


_sc_pass1 = _make_sc_pass(compute_deg=True)
_sc_pass2 = _make_sc_pass(compute_deg=False)


def _tc_body(x_ref, p_ref, d0_ref, d1_ref, ws_ref, wn_ref,
             b_ref, g_ref, be_ref, o_ref):
  deg = d0_ref[...] + d1_ref[...]
  degc = jnp.maximum(deg, 1.0)
  mean = (p_ref[0, :N_NODES, :] + p_ref[1, :N_NODES, :]) / degc
  h = (jnp.dot(x_ref[...], ws_ref[...], preferred_element_type=jnp.float32)
       + jnp.dot(mean, wn_ref[...], preferred_element_type=jnp.float32)
       + b_ref[...])
  m = jnp.mean(h, axis=0, keepdims=True)
  v = jnp.mean((h - m) * (h - m), axis=0, keepdims=True)
  hn = (h - m) * jax.lax.rsqrt(v + 1e-5) * g_ref[...] + be_ref[...]
  o_ref[...] = jnp.where(hn >= 0.0, hn, 0.01 * hn)


def _tc_layer(x, p, d0, d1, w_self, w_neigh, b, g, be):
  return pl.pallas_call(
      _tc_body,
      out_shape=jax.ShapeDtypeStruct((N_NODES, DIM), jnp.float32),
  )(x, p, d0, d1, w_self, w_neigh,
    b.reshape(1, DIM), g.reshape(1, DIM), be.reshape(1, DIM))


def kernel(x, edge_index, W1_self, W1_neigh, b1, g1, be1,
           W2_self, W2_neigh, b2, g2, be2):
  ei = jnp.stack([
      edge_index[0].astype(jnp.int32).reshape(NW, CHUNKS_PER_TILE, CK),
      edge_index[1].astype(jnp.int32).reshape(NW, CHUNKS_PER_TILE, CK),
  ], axis=2)

  acc1, degp = _sc_pass1(x, ei)
  d0 = degp[0, :N_NODES, 0:1]
  d1 = degp[1, :N_NODES, 0:1]
  h1 = _tc_layer(x, acc1, d0, d1, W1_self, W1_neigh, b1, g1, be1)
  acc2, = _sc_pass2(h1, ei)
  h2 = _tc_layer(h1, acc2, d0, d1, W2_self, W2_neigh, b2, g2, be2)
  return h2
